# baseline (speedup 1.0000x reference)
import functools

import jax
import jax.numpy as jnp
from jax.experimental import pallas as pl

_S = 8
_DSUB = 32
_K = 256
_NBLK = 4096


def _pq_body(x_ref, cb_ref, cbaug_ref, out_ref):
    for s in range(_S):
        xs = x_ref[s * _DSUB:(s + 1) * _DSUB, :]
        xaug = jnp.concatenate(
            [xs, jnp.ones((1, _NBLK), jnp.float32)], axis=0)
        scores = jax.lax.dot_general(
            cbaug_ref[s], xaug, (((0,), (0,)), ((), ())),
            preferred_element_type=jnp.float32)
        minval = jnp.min(scores, axis=0)
        onehot = (scores == minval[None, :]).astype(jnp.float32)
        out_ref[s * _DSUB:(s + 1) * _DSUB, :] = jax.lax.dot_general(
            cb_ref[s], onehot, (((1,), (0,)), ((), ())),
            preferred_element_type=jnp.float32)


@functools.partial(jax.jit, static_argnames=())
def kernel(x, codebook):
    D, N = x.shape
    cbm2 = codebook * -2.0
    c2 = jnp.sum(codebook * codebook, axis=1)
    cbaug = jnp.concatenate([cbm2, c2[:, None, :]], axis=1)
    grid = (N // _NBLK,)
    return pl.pallas_call(
        _pq_body,
        grid=grid,
        in_specs=[
            pl.BlockSpec((D, _NBLK), lambda i: (0, i)),
            pl.BlockSpec((_S, _DSUB, _K), lambda i: (0, 0, 0)),
            pl.BlockSpec((_S, _DSUB + 1, _K), lambda i: (0, 0, 0)),
        ],
        out_specs=pl.BlockSpec((D, _NBLK), lambda i: (0, i)),
        out_shape=jax.ShapeDtypeStruct((D, N), jnp.float32),
    )(x, codebook, cbaug)


# speedup vs baseline: 1.0274x; 1.0274x over previous
import functools

import jax
import jax.numpy as jnp
from jax.experimental import pallas as pl

_S = 8
_DSUB = 32
_K = 256
_NBLK = 4096


def _pq_body(x_ref, cb_ref, out_ref):
    for s in range(_S):
        xs = x_ref[s * _DSUB:(s + 1) * _DSUB, :]
        cb = cb_ref[s]
        c2 = jnp.sum(cb * cb, axis=0)
        prod = jax.lax.dot_general(
            cb * -2.0, xs, (((0,), (0,)), ((), ())),
            preferred_element_type=jnp.float32)
        scores = prod + c2[:, None]
        minval = jnp.min(scores, axis=0)
        onehot = (scores == minval[None, :]).astype(jnp.float32)
        out_ref[s * _DSUB:(s + 1) * _DSUB, :] = jax.lax.dot_general(
            cb, onehot, (((1,), (0,)), ((), ())),
            preferred_element_type=jnp.float32)


@functools.partial(jax.jit, static_argnames=())
def kernel(x, codebook):
    D, N = x.shape
    grid = (N // _NBLK,)
    return pl.pallas_call(
        _pq_body,
        grid=grid,
        in_specs=[
            pl.BlockSpec((D, _NBLK), lambda i: (0, i)),
            pl.BlockSpec((_S, _DSUB, _K), lambda i: (0, 0, 0)),
        ],
        out_specs=pl.BlockSpec((D, _NBLK), lambda i: (0, i)),
        out_shape=jax.ShapeDtypeStruct((D, N), jnp.float32),
    )(x, codebook)
